# baseline (speedup 1.0000x reference)
import functools

import jax
import jax.numpy as jnp
from jax import lax
from jax.experimental import pallas as pl
from jax.experimental.pallas import tpu as pltpu
from jax.experimental.pallas import tpu_sc as plsc

_EPS = 1e-12


def _sc_gather(table, ids, n_tokens, h):
    info = plsc.get_sparse_core_info()
    nc, ns = info.num_cores, info.num_subcores
    nw = nc * ns
    per_w = n_tokens // nw
    chunk = 128
    n_chunks = per_w // chunk
    mesh = plsc.VectorSubcoreMesh(core_axis_name="c", subcore_axis_name="s")

    @functools.partial(
        pl.kernel,
        mesh=mesh,
        out_type=jax.ShapeDtypeStruct((n_tokens, h), jnp.int32),
        scratch_types=[
            pltpu.VMEM((chunk,), jnp.int32),
            pltpu.VMEM((chunk, h), jnp.int32),
            pltpu.SemaphoreType.DMA,
        ],
    )
    def k(table_hbm, idx_hbm, out_hbm, idx_v, rows_v, sem):
        wid = lax.axis_index("s") * nc + lax.axis_index("c")
        base0 = wid * per_w

        def body(c, carry):
            base = base0 + c * chunk
            pltpu.sync_copy(idx_hbm.at[pl.ds(base, chunk)], idx_v)
            pltpu.async_copy(table_hbm.at[idx_v], rows_v, sem).wait()
            pltpu.sync_copy(rows_v, out_hbm.at[pl.ds(base, chunk)])
            return carry

        lax.fori_loop(0, n_chunks, body, 0)

    return k(table, ids)


def _tc_add_ln(emb, pos, gamma, beta):
    n_tokens, h = emb.shape
    t = 512
    grid = n_tokens // t

    def body(a_ref, b_ref, g_ref, bt_ref, o_ref):
        x = a_ref[...] + b_ref[...].astype(jnp.float32)
        mean = jnp.mean(x, axis=-1, keepdims=True)
        xc = x - mean
        var = jnp.mean(xc * xc, axis=-1, keepdims=True)
        inv = lax.rsqrt(var + _EPS)
        o_ref[...] = xc * inv * g_ref[...] + bt_ref[...]

    return pl.pallas_call(
        body,
        grid=(grid,),
        in_specs=[
            pl.BlockSpec((t, h), lambda i: (i, 0)),
            pl.BlockSpec((t, h), lambda i: (i, 0)),
            pl.BlockSpec((1, h), lambda i: (0, 0)),
            pl.BlockSpec((1, h), lambda i: (0, 0)),
        ],
        out_specs=pl.BlockSpec((t, h), lambda i: (i, 0)),
        out_shape=jax.ShapeDtypeStruct((n_tokens, h), jnp.float32),
    )(emb, pos, gamma.reshape(1, h), beta.reshape(1, h))


def kernel(input_embeds, position_ids, pos_table, ln_gamma, ln_beta):
    b, l, h = input_embeds.shape
    n = b * l
    hw = h // 2
    ids = position_ids.reshape(n).astype(jnp.int32)
    emb = input_embeds.reshape(n, h)
    table_bf = pos_table.astype(jnp.bfloat16)
    table_i = lax.bitcast_convert_type(
        table_bf.reshape(pos_table.shape[0], hw, 2), jnp.int32
    )
    rows_i = _sc_gather(table_i, ids, n, hw)
    rows_bf = lax.bitcast_convert_type(rows_i, jnp.bfloat16).reshape(n, h)
    out = _tc_add_ln(emb, rows_bf, ln_gamma, ln_beta)
    return out.reshape(b, l, h)


# speedup vs baseline: 3.7309x; 3.7309x over previous
import functools

import jax
import jax.numpy as jnp
from jax import lax
from jax.experimental import pallas as pl
from jax.experimental.pallas import tpu as pltpu
from jax.experimental.pallas import tpu_sc as plsc

_EPS = 1e-12


def _sc_gather(table, ids, n_tokens, h):
    info = plsc.get_sparse_core_info()
    nc, ns = info.num_cores, info.num_subcores
    nw = nc * ns
    per_w = n_tokens // nw
    chunk = 128
    n_chunks = per_w // chunk
    mesh = plsc.VectorSubcoreMesh(core_axis_name="c", subcore_axis_name="s")

    @functools.partial(
        pl.kernel,
        mesh=mesh,
        out_type=jax.ShapeDtypeStruct((n_tokens, h), jnp.int32),
        scratch_types=[
            pltpu.VMEM((chunk,), jnp.int32),
            pltpu.VMEM((chunk, h), jnp.int32),
            pltpu.SemaphoreType.DMA,
        ],
    )
    def k(table_hbm, idx_hbm, out_hbm, idx_v, rows_v, sem):
        wid = lax.axis_index("s") * nc + lax.axis_index("c")
        base0 = wid * per_w

        def body(c, carry):
            base = base0 + c * chunk
            pltpu.sync_copy(idx_hbm.at[pl.ds(base, chunk)], idx_v)
            pltpu.async_copy(table_hbm.at[idx_v], rows_v, sem).wait()
            pltpu.sync_copy(rows_v, out_hbm.at[pl.ds(base, chunk)])
            return carry

        lax.fori_loop(0, n_chunks, body, 0)

    return k(table, ids)


def _tc_add_ln(emb, pos, gamma, beta):
    n_tokens, h = emb.shape
    t = 512
    grid = n_tokens // t

    hw = h // 2

    def body(a_ref, b_ref, g_ref, bt_ref, o_ref):
        w = b_ref[...]
        lo = lax.bitcast_convert_type(lax.shift_left(w, 16), jnp.float32)
        hi = lax.bitcast_convert_type(
            jnp.bitwise_and(w, jnp.int32(-65536)), jnp.float32
        )
        x = a_ref[...] + jnp.concatenate([lo, hi], axis=-1)
        mean = jnp.mean(x, axis=-1, keepdims=True)
        xc = x - mean
        var = jnp.mean(xc * xc, axis=-1, keepdims=True)
        inv = lax.rsqrt(var + _EPS)
        o_ref[...] = xc * inv * g_ref[...] + bt_ref[...]

    return pl.pallas_call(
        body,
        grid=(grid,),
        in_specs=[
            pl.BlockSpec((t, h), lambda i: (i, 0)),
            pl.BlockSpec((t, hw), lambda i: (i, 0)),
            pl.BlockSpec((1, h), lambda i: (0, 0)),
            pl.BlockSpec((1, h), lambda i: (0, 0)),
        ],
        out_specs=pl.BlockSpec((t, h), lambda i: (i, 0)),
        out_shape=jax.ShapeDtypeStruct((n_tokens, h), jnp.float32),
    )(emb, pos, gamma.reshape(1, h), beta.reshape(1, h))


def kernel(input_embeds, position_ids, pos_table, ln_gamma, ln_beta):
    b, l, h = input_embeds.shape
    n = b * l
    hw = h // 2
    ids = position_ids.reshape(n).astype(jnp.int32)
    emb = input_embeds.reshape(n, h)
    table_bf = pos_table.astype(jnp.bfloat16)
    lo = lax.bitcast_convert_type(table_bf[:, :hw], jnp.uint16).astype(jnp.uint32)
    hi = lax.bitcast_convert_type(table_bf[:, hw:], jnp.uint16).astype(jnp.uint32)
    table_i = lax.bitcast_convert_type(lo | (hi << 16), jnp.int32)
    rows_i = _sc_gather(table_i, ids, n, hw)
    out = _tc_add_ln(emb, rows_i, ln_gamma, ln_beta)
    return out.reshape(b, l, h)


# speedup vs baseline: 4.0674x; 1.0902x over previous
import functools

import jax
import jax.numpy as jnp
from jax import lax
from jax.experimental import pallas as pl
from jax.experimental.pallas import tpu as pltpu
from jax.experimental.pallas import tpu_sc as plsc

_EPS = 1e-12


def _sc_gather(table, ids, n_tokens, h):
    info = plsc.get_sparse_core_info()
    nc, ns = info.num_cores, info.num_subcores
    nw = nc * ns
    per_w = n_tokens // nw
    chunk = 128
    n_chunks = per_w // chunk
    mesh = plsc.VectorSubcoreMesh(core_axis_name="c", subcore_axis_name="s")

    @functools.partial(
        pl.kernel,
        mesh=mesh,
        out_type=jax.ShapeDtypeStruct((n_tokens, h), jnp.int32),
        scratch_types=[
            pltpu.VMEM((chunk,), jnp.int32),
            pltpu.VMEM((chunk, h), jnp.int32),
            pltpu.SemaphoreType.DMA,
        ],
    )
    def k(table_hbm, idx_hbm, out_hbm, idx_v, rows_v, sem):
        wid = lax.axis_index("s") * nc + lax.axis_index("c")
        base0 = wid * per_w

        def body(c, carry):
            base = base0 + c * chunk
            pltpu.sync_copy(idx_hbm.at[pl.ds(base, chunk)], idx_v)
            pltpu.async_copy(table_hbm.at[idx_v], rows_v, sem).wait()
            pltpu.sync_copy(rows_v, out_hbm.at[pl.ds(base, chunk)])
            return carry

        lax.fori_loop(0, n_chunks, body, 0)

    return k(table, ids)


def _tc_add_ln(emb, pos, gamma, beta):
    n_tokens, h = emb.shape
    t = 1024
    grid = n_tokens // t

    hw = h // 2

    def body(a_ref, b_ref, g_ref, bt_ref, o_ref):
        w = b_ref[...]
        lo = lax.bitcast_convert_type(lax.shift_left(w, 16), jnp.float32)
        hi = lax.bitcast_convert_type(
            jnp.bitwise_and(w, jnp.int32(-65536)), jnp.float32
        )
        x = a_ref[...] + jnp.concatenate([lo, hi], axis=-1)
        mean = jnp.mean(x, axis=-1, keepdims=True)
        xc = x - mean
        var = jnp.mean(xc * xc, axis=-1, keepdims=True)
        inv = lax.rsqrt(var + _EPS)
        o_ref[...] = xc * inv * g_ref[...] + bt_ref[...]

    return pl.pallas_call(
        body,
        grid=(grid,),
        in_specs=[
            pl.BlockSpec((t, h), lambda i: (i, 0)),
            pl.BlockSpec((t, hw), lambda i: (i, 0)),
            pl.BlockSpec((1, h), lambda i: (0, 0)),
            pl.BlockSpec((1, h), lambda i: (0, 0)),
        ],
        out_specs=pl.BlockSpec((t, h), lambda i: (i, 0)),
        out_shape=jax.ShapeDtypeStruct((n_tokens, h), jnp.float32),
    )(emb, pos, gamma.reshape(1, h), beta.reshape(1, h))


def kernel(input_embeds, position_ids, pos_table, ln_gamma, ln_beta):
    b, l, h = input_embeds.shape
    n = b * l
    hw = h // 2
    ids = position_ids.reshape(n).astype(jnp.int32)
    emb = input_embeds.reshape(n, h)
    table_bf = pos_table.astype(jnp.bfloat16)
    lo = lax.bitcast_convert_type(table_bf[:, :hw], jnp.uint16).astype(jnp.uint32)
    hi = lax.bitcast_convert_type(table_bf[:, hw:], jnp.uint16).astype(jnp.uint32)
    table_i = lax.bitcast_convert_type(lo | (hi << 16), jnp.int32)
    rows_i = _sc_gather(table_i, ids, n, hw)
    out = _tc_add_ln(emb, rows_i, ln_gamma, ln_beta)
    return out.reshape(b, l, h)


# speedup vs baseline: 4.1453x; 1.0191x over previous
import functools

import jax
import jax.numpy as jnp
from jax import lax
from jax.experimental import pallas as pl
from jax.experimental.pallas import tpu as pltpu
from jax.experimental.pallas import tpu_sc as plsc

_EPS = 1e-12


def _sc_gather(table, ids, n_tokens, h):
    info = plsc.get_sparse_core_info()
    nc, ns = info.num_cores, info.num_subcores
    nw = nc * ns
    per_w = n_tokens // nw
    chunk = 128
    n_chunks = per_w // chunk
    mesh = plsc.VectorSubcoreMesh(core_axis_name="c", subcore_axis_name="s")

    @functools.partial(
        pl.kernel,
        mesh=mesh,
        out_type=jax.ShapeDtypeStruct((n_tokens, h), jnp.int32),
        scratch_types=[
            pltpu.VMEM((chunk,), jnp.int32),
            pltpu.VMEM((chunk, h), jnp.int32),
            pltpu.SemaphoreType.DMA,
        ],
    )
    def k(table_hbm, idx_hbm, out_hbm, idx_v, rows_v, sem):
        wid = lax.axis_index("s") * nc + lax.axis_index("c")
        base0 = wid * per_w

        def body(c, carry):
            base = base0 + c * chunk
            pltpu.sync_copy(idx_hbm.at[pl.ds(base, chunk)], idx_v)
            pltpu.async_copy(table_hbm.at[idx_v], rows_v, sem).wait()
            pltpu.sync_copy(rows_v, out_hbm.at[pl.ds(base, chunk)])
            return carry

        lax.fori_loop(0, n_chunks, body, 0)

    return k(table, ids)


def _tc_add_ln(emb, pos, gamma, beta):
    n_tokens, h = emb.shape
    t = 2048
    grid = n_tokens // t

    hw = h // 2

    def body(a_ref, b_ref, g_ref, bt_ref, o_ref):
        w = b_ref[...]
        lo = lax.bitcast_convert_type(lax.shift_left(w, 16), jnp.float32)
        hi = lax.bitcast_convert_type(
            jnp.bitwise_and(w, jnp.int32(-65536)), jnp.float32
        )
        x = a_ref[...] + jnp.concatenate([lo, hi], axis=-1)
        mean = jnp.mean(x, axis=-1, keepdims=True)
        xc = x - mean
        var = jnp.mean(xc * xc, axis=-1, keepdims=True)
        inv = lax.rsqrt(var + _EPS)
        o_ref[...] = xc * inv * g_ref[...] + bt_ref[...]

    return pl.pallas_call(
        body,
        grid=(grid,),
        in_specs=[
            pl.BlockSpec((t, h), lambda i: (i, 0)),
            pl.BlockSpec((t, hw), lambda i: (i, 0)),
            pl.BlockSpec((1, h), lambda i: (0, 0)),
            pl.BlockSpec((1, h), lambda i: (0, 0)),
        ],
        out_specs=pl.BlockSpec((t, h), lambda i: (i, 0)),
        out_shape=jax.ShapeDtypeStruct((n_tokens, h), jnp.float32),
    )(emb, pos, gamma.reshape(1, h), beta.reshape(1, h))


def kernel(input_embeds, position_ids, pos_table, ln_gamma, ln_beta):
    b, l, h = input_embeds.shape
    n = b * l
    hw = h // 2
    ids = position_ids.reshape(n).astype(jnp.int32)
    emb = input_embeds.reshape(n, h)
    table_bf = pos_table.astype(jnp.bfloat16)
    lo = lax.bitcast_convert_type(table_bf[:, :hw], jnp.uint16).astype(jnp.uint32)
    hi = lax.bitcast_convert_type(table_bf[:, hw:], jnp.uint16).astype(jnp.uint32)
    table_i = lax.bitcast_convert_type(lo | (hi << 16), jnp.int32)
    rows_i = _sc_gather(table_i, ids, n, hw)
    out = _tc_add_ln(emb, rows_i, ln_gamma, ln_beta)
    return out.reshape(b, l, h)
